# Initial kernel scaffold; baseline (speedup 1.0000x reference)
import jax
import jax.numpy as jnp
from jax.experimental import pallas as pl


def kernel(x, q_id, k_id, Wq, bq, Wk, bk, Wv, bv, Wx, bx):
    raise NotImplementedError("write your pallas kernel here")


# speedup vs baseline: 8.7591x; 8.7591x over previous
import functools

import jax
import jax.numpy as jnp
from jax import lax
from jax.experimental import pallas as pl
from jax.experimental.pallas import tpu as pltpu

S = 2048
C = 2048
NH = 32
CC = C // NH


def _matmul_bias_kernel(x_ref, w_ref, b_ref, o_ref):
    o_ref[...] = (
        jnp.dot(x_ref[...], w_ref[...], preferred_element_type=jnp.float32)
        + b_ref[...]
    )


def _matmul_bias(x, w, b, bm=256, bn=512):
    m, k = x.shape
    _, n = w.shape
    grid = (m // bm, n // bn)
    return pl.pallas_call(
        _matmul_bias_kernel,
        grid=grid,
        in_specs=[
            pl.BlockSpec((bm, k), lambda i, j: (i, 0)),
            pl.BlockSpec((k, bn), lambda i, j: (0, j)),
            pl.BlockSpec((1, bn), lambda i, j: (0, j)),
        ],
        out_specs=pl.BlockSpec((bm, bn), lambda i, j: (i, j)),
        out_shape=jax.ShapeDtypeStruct((m, n), jnp.float32),
        compiler_params=pltpu.CompilerParams(
            dimension_semantics=("parallel", "parallel"),
        ),
    )(x, w, b.reshape(1, n))


def _flash_kernel(q_ref, k_ref, v_ref, mask_ref, o_ref, m_s, l_s, acc):
    j = pl.program_id(1)
    nk = pl.num_programs(1)

    @pl.when(j == 0)
    def _init():
        m_s[...] = jnp.full_like(m_s, -1e30)
        l_s[...] = jnp.zeros_like(l_s)
        acc[...] = jnp.zeros_like(acc)

    q = q_ref[...]
    k = k_ref[...]
    v = v_ref[...]
    msk = mask_ref[...]

    s = lax.dot_general(
        q, k, (((2,), (2,)), ((0,), (0,))),
        preferred_element_type=jnp.float32,
    )

    m_prev = m_s[...]
    m_new = jnp.maximum(m_prev, jnp.max(s, axis=-1))
    alpha = jnp.exp(m_prev - m_new)
    p = jnp.exp(s - m_new[..., None]) * msk[None, :, :]
    l_s[...] = l_s[...] * alpha + jnp.sum(p, axis=-1)
    acc[...] = acc[...] * alpha[..., None] + lax.dot_general(
        p, v, (((2,), (1,)), ((0,), (0,))),
        preferred_element_type=jnp.float32,
    )
    m_s[...] = m_new

    @pl.when(j == nk - 1)
    def _done():
        o_ref[...] = acc[...] / l_s[...][..., None]


def _flash_attention(qh, kh, vh, mask, bq=256, bk=256):
    grid = (S // bq, S // bk)
    return pl.pallas_call(
        _flash_kernel,
        grid=grid,
        in_specs=[
            pl.BlockSpec((NH, bq, CC), lambda i, j: (0, i, 0)),
            pl.BlockSpec((NH, bk, CC), lambda i, j: (0, j, 0)),
            pl.BlockSpec((NH, bk, CC), lambda i, j: (0, j, 0)),
            pl.BlockSpec((bq, bk), lambda i, j: (i, j)),
        ],
        out_specs=pl.BlockSpec((NH, bq, CC), lambda i, j: (0, i, 0)),
        out_shape=jax.ShapeDtypeStruct((NH, S, CC), jnp.float32),
        scratch_shapes=[
            pltpu.VMEM((NH, bq), jnp.float32),
            pltpu.VMEM((NH, bq), jnp.float32),
            pltpu.VMEM((NH, bq, CC), jnp.float32),
        ],
        compiler_params=pltpu.CompilerParams(
            dimension_semantics=("parallel", "arbitrary"),
        ),
    )(qh, kh, vh, mask)


def _build_mask(q_id, k_id):
    return jnp.zeros((S, S), jnp.float32).at[q_id, k_id].add(1.0)


def kernel(x, q_id, k_id, Wq, bq, Wk, bk, Wv, bv, Wx, bx):
    s, b, c = x.shape
    x2 = x.reshape(S, C)

    scale = 1.0 / (CC ** 0.5)
    w_cat = jnp.concatenate([Wq * scale, Wk, Wv], axis=1)
    b_cat = jnp.concatenate([bq * scale, bk, bv])

    qkv = _matmul_bias(x2, w_cat, b_cat)
    q2, k2, v2 = qkv[:, :C], qkv[:, C:2 * C], qkv[:, 2 * C:]
    qh = q2.reshape(S, NH, CC).transpose(1, 0, 2)
    kh = k2.reshape(S, NH, CC).transpose(1, 0, 2)
    vh = v2.reshape(S, NH, CC).transpose(1, 0, 2)

    mask = _build_mask(q_id, k_id)

    oh = _flash_attention(qh, kh, vh, mask)
    o2 = oh.transpose(1, 0, 2).reshape(S, C)

    out = _matmul_bias(o2, Wx, bx)
    return out.reshape(s, b, c)


# speedup vs baseline: 9.3229x; 1.0644x over previous
import functools

import jax
import jax.numpy as jnp
from jax import lax
from jax.experimental import pallas as pl
from jax.experimental.pallas import tpu as pltpu
from jax.experimental.pallas import tpu_sc as plsc

S = 2048
C = 2048
NH = 32
CC = C // NH


def _matmul_bias_kernel(x_ref, w_ref, b_ref, o_ref):
    o_ref[...] = (
        jnp.dot(x_ref[...], w_ref[...], preferred_element_type=jnp.float32)
        + b_ref[...]
    )


def _matmul_bias(x, w, b, bm=256, bn=512):
    m, k = x.shape
    _, n = w.shape
    grid = (m // bm, n // bn)
    return pl.pallas_call(
        _matmul_bias_kernel,
        grid=grid,
        in_specs=[
            pl.BlockSpec((bm, k), lambda i, j: (i, 0)),
            pl.BlockSpec((k, bn), lambda i, j: (0, j)),
            pl.BlockSpec((1, bn), lambda i, j: (0, j)),
        ],
        out_specs=pl.BlockSpec((bm, bn), lambda i, j: (i, j)),
        out_shape=jax.ShapeDtypeStruct((m, n), jnp.float32),
        compiler_params=pltpu.CompilerParams(
            dimension_semantics=("parallel", "parallel"),
        ),
    )(x, w, b.reshape(1, n))


def _flash_kernel(q_ref, k_ref, v_ref, mask_ref, o_ref, m_s, l_s, acc):
    j = pl.program_id(1)
    nk = pl.num_programs(1)

    @pl.when(j == 0)
    def _init():
        m_s[...] = jnp.full_like(m_s, -1e30)
        l_s[...] = jnp.zeros_like(l_s)
        acc[...] = jnp.zeros_like(acc)

    q = q_ref[...]
    k = k_ref[...]
    v = v_ref[...]
    msk = mask_ref[...]

    s = lax.dot_general(
        q, k, (((2,), (2,)), ((0,), (0,))),
        preferred_element_type=jnp.float32,
    )

    m_prev = m_s[...]
    m_new = jnp.maximum(m_prev, jnp.max(s, axis=-1))
    alpha = jnp.exp(m_prev - m_new)
    p = jnp.exp(s - m_new[..., None]) * msk[None, :, :]
    l_s[...] = l_s[...] * alpha + jnp.sum(p, axis=-1)
    acc[...] = acc[...] * alpha[..., None] + lax.dot_general(
        p, v, (((2,), (1,)), ((0,), (0,))),
        preferred_element_type=jnp.float32,
    )
    m_s[...] = m_new

    @pl.when(j == nk - 1)
    def _done():
        o_ref[...] = acc[...] / l_s[...][..., None]


def _flash_attention(qh, kh, vh, mask, bq=256, bk=256):
    grid = (S // bq, S // bk)
    return pl.pallas_call(
        _flash_kernel,
        grid=grid,
        in_specs=[
            pl.BlockSpec((NH, bq, CC), lambda i, j: (0, i, 0)),
            pl.BlockSpec((NH, bk, CC), lambda i, j: (0, j, 0)),
            pl.BlockSpec((NH, bk, CC), lambda i, j: (0, j, 0)),
            pl.BlockSpec((bq, bk), lambda i, j: (i, j)),
        ],
        out_specs=pl.BlockSpec((NH, bq, CC), lambda i, j: (0, i, 0)),
        out_shape=jax.ShapeDtypeStruct((NH, S, CC), jnp.float32),
        scratch_shapes=[
            pltpu.VMEM((NH, bq), jnp.float32),
            pltpu.VMEM((NH, bq), jnp.float32),
            pltpu.VMEM((NH, bq, CC), jnp.float32),
        ],
        compiler_params=pltpu.CompilerParams(
            dimension_semantics=("parallel", "arbitrary"),
        ),
    )(qh, kh, vh, mask)


_NC = 2
_NS = 16
_NW = _NC * _NS
_ROWS = S // (_NW * 2)
_BLK = _ROWS * S
_ECH = 2048
_NCH = 32768 // _ECH


def _sc_mask_body(qid_hbm, kid_hbm, out_hbm, qv, kv, acc):
    wid = lax.axis_index("s") * _NC + lax.axis_index("c")
    ones = jnp.full((16,), 1.0, jnp.float32)
    zeros = jnp.zeros((16,), jnp.float32)
    for rnd in range(2):
        blk = rnd * _NW + wid
        base = blk * _ROWS

        def _zero(i, _):
            acc[pl.ds(pl.multiple_of(i * 16, 16), 16)] = zeros
            return 0

        lax.fori_loop(0, _BLK // 16, _zero, 0)

        for ch in range(_NCH):
            pltpu.sync_copy(qid_hbm.at[pl.ds(ch * _ECH, _ECH)], qv)
            pltpu.sync_copy(kid_hbm.at[pl.ds(ch * _ECH, _ECH)], kv)

            def _scat(i, _):
                off = pl.ds(pl.multiple_of(i * 16, 16), 16)
                q16 = qv[off]
                k16 = kv[off]
                msk = (q16 >= base) & (q16 < base + _ROWS)
                rel = jnp.where(msk, (q16 - base) * S + k16, 0)
                plsc.addupdate_scatter(acc, [rel], ones, mask=msk)
                return 0

            lax.fori_loop(0, _ECH // 16, _scat, 0)

        pltpu.sync_copy(
            acc, out_hbm.at[pl.ds(pl.multiple_of(blk * _BLK, _BLK), _BLK)]
        )


def _build_mask(q_id, k_id):
    f = functools.partial(
        pl.kernel,
        mesh=plsc.VectorSubcoreMesh(core_axis_name="c", subcore_axis_name="s"),
        out_type=jax.ShapeDtypeStruct((S * S,), jnp.float32),
        scratch_types=[
            pltpu.VMEM((_ECH,), jnp.int32),
            pltpu.VMEM((_ECH,), jnp.int32),
            pltpu.VMEM((_BLK,), jnp.float32),
        ],
        compiler_params=pltpu.CompilerParams(needs_layout_passes=False),
    )(_sc_mask_body)
    return f(q_id, k_id).reshape(S, S)


def kernel(x, q_id, k_id, Wq, bq, Wk, bk, Wv, bv, Wx, bx):
    s, b, c = x.shape
    x2 = x.reshape(S, C)

    scale = 1.0 / (CC ** 0.5)
    w_cat = jnp.concatenate([Wq * scale, Wk, Wv], axis=1)
    b_cat = jnp.concatenate([bq * scale, bk, bv])

    qkv = _matmul_bias(x2, w_cat, b_cat)
    q2, k2, v2 = qkv[:, :C], qkv[:, C:2 * C], qkv[:, 2 * C:]
    qh = q2.reshape(S, NH, CC).transpose(1, 0, 2)
    kh = k2.reshape(S, NH, CC).transpose(1, 0, 2)
    vh = v2.reshape(S, NH, CC).transpose(1, 0, 2)

    mask = _build_mask(q_id, k_id)

    oh = _flash_attention(qh, kh, vh, mask)
    o2 = oh.transpose(1, 0, 2).reshape(S, C)

    out = _matmul_bias(o2, Wx, bx)
    return out.reshape(s, b, c)


# speedup vs baseline: 10.9185x; 1.1712x over previous
import functools

import jax
import jax.numpy as jnp
from jax import lax
from jax.experimental import pallas as pl
from jax.experimental.pallas import tpu as pltpu
from jax.experimental.pallas import tpu_sc as plsc

S = 2048
C = 2048
NH = 32
CC = C // NH


def _matmul_bias_kernel(x_ref, w_ref, b_ref, o_ref):
    o_ref[...] = (
        jnp.dot(x_ref[...], w_ref[...], preferred_element_type=jnp.float32)
        + b_ref[...]
    )


def _matmul_bias(x, w, b, bm=256, bn=512):
    m, k = x.shape
    _, n = w.shape
    grid = (m // bm, n // bn)
    return pl.pallas_call(
        _matmul_bias_kernel,
        grid=grid,
        in_specs=[
            pl.BlockSpec((bm, k), lambda i, j: (i, 0)),
            pl.BlockSpec((k, bn), lambda i, j: (0, j)),
            pl.BlockSpec((1, bn), lambda i, j: (0, j)),
        ],
        out_specs=pl.BlockSpec((bm, bn), lambda i, j: (i, j)),
        out_shape=jax.ShapeDtypeStruct((m, n), jnp.float32),
        compiler_params=pltpu.CompilerParams(
            dimension_semantics=("parallel", "parallel"),
        ),
    )(x, w, b.reshape(1, n))


def _proj_head_kernel(scale, x_ref, w_ref, b_ref, o_ref):
    y = (
        jnp.dot(x_ref[...], w_ref[...], preferred_element_type=jnp.float32)
        + b_ref[...]
    ) * scale
    bm, bn = y.shape
    o_ref[...] = jnp.swapaxes(y.reshape(bm, bn // CC, CC), 0, 1)


def _proj_heads(x, w, b, scale=1.0, bm=256, bn=512):
    grid = (S // bm, C // bn)
    hb = bn // CC
    return pl.pallas_call(
        functools.partial(_proj_head_kernel, scale),
        grid=grid,
        in_specs=[
            pl.BlockSpec((bm, C), lambda i, j: (i, 0)),
            pl.BlockSpec((C, bn), lambda i, j: (0, j)),
            pl.BlockSpec((1, bn), lambda i, j: (0, j)),
        ],
        out_specs=pl.BlockSpec((hb, bm, CC), lambda i, j: (j, i, 0)),
        out_shape=jax.ShapeDtypeStruct((NH, S, CC), jnp.float32),
        compiler_params=pltpu.CompilerParams(
            dimension_semantics=("parallel", "parallel"),
        ),
    )(x, w, b.reshape(1, C))


def _flash_kernel(q_ref, k_ref, v_ref, mask_ref, o_ref, m_s, l_s, acc):
    j = pl.program_id(1)
    nk = pl.num_programs(1)

    @pl.when(j == 0)
    def _init():
        m_s[...] = jnp.full_like(m_s, -1e30)
        l_s[...] = jnp.zeros_like(l_s)
        acc[...] = jnp.zeros_like(acc)

    q = q_ref[...]
    k = k_ref[...]
    v = v_ref[...]
    msk = mask_ref[...]

    s = lax.dot_general(
        q, k, (((2,), (2,)), ((0,), (0,))),
        preferred_element_type=jnp.float32,
    )

    m_prev = m_s[...]
    m_new = jnp.maximum(m_prev, jnp.max(s, axis=-1))
    alpha = jnp.exp(m_prev - m_new)
    p = jnp.exp(s - m_new[..., None]) * msk[None, :, :]
    l_s[...] = l_s[...] * alpha + jnp.sum(p, axis=-1)
    acc[...] = acc[...] * alpha[..., None] + lax.dot_general(
        p, v, (((2,), (1,)), ((0,), (0,))),
        preferred_element_type=jnp.float32,
    )
    m_s[...] = m_new

    @pl.when(j == nk - 1)
    def _done():
        o = acc[...] / l_s[...][..., None]
        bq = o.shape[1]
        o_ref[...] = jnp.swapaxes(o, 0, 1).reshape(bq, C)


def _flash_attention(qh, kh, vh, mask, bq=256, bk=256):
    grid = (S // bq, S // bk)
    return pl.pallas_call(
        _flash_kernel,
        grid=grid,
        in_specs=[
            pl.BlockSpec((NH, bq, CC), lambda i, j: (0, i, 0)),
            pl.BlockSpec((NH, bk, CC), lambda i, j: (0, j, 0)),
            pl.BlockSpec((NH, bk, CC), lambda i, j: (0, j, 0)),
            pl.BlockSpec((bq, bk), lambda i, j: (i, j)),
        ],
        out_specs=pl.BlockSpec((bq, C), lambda i, j: (i, 0)),
        out_shape=jax.ShapeDtypeStruct((S, C), jnp.float32),
        scratch_shapes=[
            pltpu.VMEM((NH, bq), jnp.float32),
            pltpu.VMEM((NH, bq), jnp.float32),
            pltpu.VMEM((NH, bq, CC), jnp.float32),
        ],
        compiler_params=pltpu.CompilerParams(
            dimension_semantics=("parallel", "arbitrary"),
        ),
    )(qh, kh, vh, mask)


_NC = 2
_NS = 16
_NW = _NC * _NS
_ROWS = S // (_NW * 2)
_BLK = _ROWS * S
_ECH = 2048
_NCH = 32768 // _ECH


def _sc_mask_body(qid_hbm, kid_hbm, out_hbm, qv, kv, acc):
    wid = lax.axis_index("s") * _NC + lax.axis_index("c")
    ones = jnp.full((16,), 1.0, jnp.float32)
    zeros = jnp.zeros((16,), jnp.float32)
    for rnd in range(2):
        blk = rnd * _NW + wid
        base = blk * _ROWS

        def _zero(i, _):
            acc[pl.ds(pl.multiple_of(i * 16, 16), 16)] = zeros
            return 0

        lax.fori_loop(0, _BLK // 16, _zero, 0)

        for ch in range(_NCH):
            pltpu.sync_copy(qid_hbm.at[pl.ds(ch * _ECH, _ECH)], qv)
            pltpu.sync_copy(kid_hbm.at[pl.ds(ch * _ECH, _ECH)], kv)

            def _scat(i, _):
                off = pl.ds(pl.multiple_of(i * 16, 16), 16)
                q16 = qv[off]
                k16 = kv[off]
                msk = (q16 >= base) & (q16 < base + _ROWS)
                rel = jnp.where(msk, (q16 - base) * S + k16, 0)
                plsc.addupdate_scatter(acc, [rel], ones, mask=msk)
                return 0

            lax.fori_loop(0, _ECH // 16, _scat, 0)

        pltpu.sync_copy(
            acc, out_hbm.at[pl.ds(pl.multiple_of(blk * _BLK, _BLK), _BLK)]
        )


def _build_mask(q_id, k_id):
    f = functools.partial(
        pl.kernel,
        mesh=plsc.VectorSubcoreMesh(core_axis_name="c", subcore_axis_name="s"),
        out_type=jax.ShapeDtypeStruct((S * S,), jnp.float32),
        scratch_types=[
            pltpu.VMEM((_ECH,), jnp.int32),
            pltpu.VMEM((_ECH,), jnp.int32),
            pltpu.VMEM((_BLK,), jnp.float32),
        ],
        compiler_params=pltpu.CompilerParams(needs_layout_passes=False),
    )(_sc_mask_body)
    return f(q_id, k_id).reshape(S, S)


def kernel(x, q_id, k_id, Wq, bq, Wk, bk, Wv, bv, Wx, bx):
    s, b, c = x.shape
    x2 = x.reshape(S, C)

    qh = _proj_heads(x2, Wq, bq, scale=1.0 / (CC ** 0.5))
    kh = _proj_heads(x2, Wk, bk)
    vh = _proj_heads(x2, Wv, bv)

    mask = _build_mask(q_id, k_id)

    o2 = _flash_attention(qh, kh, vh, mask)
    out = _matmul_bias(o2, Wx, bx)
    return out.reshape(s, b, c)
